# baseline (speedup 1.0000x reference)
import functools
import math

import jax
import jax.numpy as jnp
from jax import lax
from jax.experimental import pallas as pl
from jax.experimental.pallas import tpu as pltpu
from jax.experimental.pallas import tpu_sc as plsc

D = 64
DP = 128
SCALE = math.sqrt(D)

_info = plsc.get_sparse_core_info()
NC, NS, L = _info.num_cores, _info.num_subcores, _info.num_lanes
NW = NC * NS

CH = 128
NBUF = 4
NWB = 2


@functools.lru_cache(maxsize=None)
def _make(s0, s1):
    assert s0 == NW * CH
    mesh = plsc.VectorSubcoreMesh(core_axis_name="c", subcore_axis_name="s")

    @functools.partial(
        pl.kernel,
        mesh=mesh,
        out_type=jax.ShapeDtypeStruct((s1, D, s0), jnp.float32),
        scratch_types=[
            pltpu.VMEM((s1, CH), jnp.int32),
            [pltpu.VMEM((CH, DP), jnp.float32) for _ in range(NBUF)],
            [pltpu.VMEM((D, CH), jnp.float32) for _ in range(NWB)],
            [pltpu.SemaphoreType.DMA for _ in range(NBUF)],
            [pltpu.SemaphoreType.DMA for _ in range(NWB)],
        ],
        compiler_params=pltpu.CompilerParams(needs_layout_passes=False),
    )
    def k(xt_hbm, table_hbm, out_hbm, idxv, gbufs, tbufs, gsems, wsems):
        wid = lax.axis_index("s") * NC + lax.axis_index("c")
        lane0 = pl.multiple_of(wid * CH, CH)

        pltpu.sync_copy(xt_hbm.at[:, pl.ds(lane0, CH)], idxv)

        rowvecs = [lax.iota(jnp.int32, L) + q * L for q in range(CH // L)]

        def g_desc(b, t):
            return pltpu.make_async_copy(
                table_hbm.at[idxv.at[t]], gbufs[b], gsems[b])

        def w_desc(b, t):
            return pltpu.make_async_copy(
                tbufs[b % NWB],
                out_hbm.at[t, :, pl.ds(lane0, CH)],
                wsems[b % NWB],
            )

        def transpose_scale(gb, tb):
            def j_body(j, c):
                colvec = jnp.full((L,), 0, jnp.int32) + j
                for q in range(CH // L):
                    v = plsc.load_gather(gb, [rowvecs[q], colvec])
                    tb[j, pl.ds(q * L, L)] = v * SCALE
                return c

            lax.fori_loop(0, D, j_body, 0, unroll=4)

        for b in range(NBUF):
            g_desc(b, b).start()

        def rnd(i, c):
            s = i * NBUF
            for b in range(NBUF):
                t = s + b
                g_desc(b, t).wait()

                @pl.when(t >= NWB)
                def _():
                    w_desc(b, t - NWB).wait()

                transpose_scale(gbufs[b], tbufs[b % NWB])

                @pl.when(t + NBUF < s1)
                def _():
                    g_desc(b, t + NBUF).start()

                w_desc(b, t).start()
            return c

        lax.fori_loop(0, s1 // NBUF, rnd, 0)

        for b in range(NWB):
            w_desc(b, s1 - NWB + b).wait()

    return k


@jax.jit
def kernel(x, embedding):
    s0, s1 = x.shape
    xt = x.T.astype(jnp.int32)
    table2 = jnp.pad(embedding, ((0, 0), (0, DP - D)))
    out_t = _make(s0, s1)(xt, table2)
    return jnp.transpose(out_t, (2, 0, 1))


# speedup vs baseline: 1.8636x; 1.8636x over previous
import functools
import math

import jax
import jax.numpy as jnp
from jax import lax
from jax.experimental import pallas as pl
from jax.experimental.pallas import tpu as pltpu
from jax.experimental.pallas import tpu_sc as plsc

D = 64
DP = 128
SCALE = math.sqrt(D)

_info = plsc.get_sparse_core_info()
NC, NS, L = _info.num_cores, _info.num_subcores, _info.num_lanes
NW = NC * NS

CH = 128
NBUF = 4
NWB = 2


@functools.lru_cache(maxsize=None)
def _make(steps):
    mesh = plsc.VectorSubcoreMesh(core_axis_name="c", subcore_axis_name="s")

    @functools.partial(
        pl.kernel,
        mesh=mesh,
        out_type=jax.ShapeDtypeStruct((NW, steps, CH, D), jnp.float32),
        scratch_types=[
            pltpu.VMEM((steps, CH), jnp.int32),
            [pltpu.VMEM((CH, DP), jnp.float32) for _ in range(NBUF)],
            [pltpu.VMEM((CH, D), jnp.float32) for _ in range(NWB)],
            [pltpu.SemaphoreType.DMA for _ in range(NBUF)],
            [pltpu.SemaphoreType.DMA for _ in range(NWB)],
        ],
        compiler_params=pltpu.CompilerParams(needs_layout_passes=False),
    )
    def k(x_hbm, table_hbm, out_hbm, idxv, gbufs, wbufs, gsems, wsems):
        wid = lax.axis_index("s") * NC + lax.axis_index("c")

        pltpu.sync_copy(x_hbm.at[wid], idxv)

        def g_desc(b, t):
            return pltpu.make_async_copy(
                table_hbm.at[idxv.at[t]], gbufs[b], gsems[b])

        def w_desc(b, t):
            return pltpu.make_async_copy(
                wbufs[b % NWB], out_hbm.at[wid, t], wsems[b % NWB])

        def scale_copy(gb, wb):
            @plsc.parallel_loop(0, CH, unroll=8)
            def _(r):
                for q in range(D // L):
                    wb[r, pl.ds(q * L, L)] = gb[r, pl.ds(q * L, L)] * SCALE

        for b in range(NBUF):
            g_desc(b, b).start()

        def rnd(i, c):
            s = i * NBUF
            for b in range(NBUF):
                t = s + b
                g_desc(b, t).wait()

                @pl.when(t >= NWB)
                def _():
                    w_desc(b, t - NWB).wait()

                scale_copy(gbufs[b], wbufs[b % NWB])

                @pl.when(t + NBUF < steps)
                def _():
                    g_desc(b, t + NBUF).start()

                w_desc(b, t).start()
            return c

        lax.fori_loop(0, steps // NBUF, rnd, 0)

        for b in range(NWB):
            w_desc(b, steps - NWB + b).wait()

    return k


@jax.jit
def kernel(x, embedding):
    s0, s1 = x.shape
    n = s0 * s1
    steps = n // (NW * CH)
    xr = x.reshape(NW, steps, CH).astype(jnp.int32)
    table2 = jnp.pad(embedding, ((0, 0), (0, DP - D)))
    out = _make(steps)(xr, table2)
    return out.reshape(s0, s1, D)
